# baseline (speedup 1.0000x reference)
import jax
import jax.numpy as jnp
from jax import lax
from jax.experimental import pallas as pl
from jax.experimental.pallas import tpu as pltpu
from jax.experimental.pallas import tpu_sc as plsc

N = 10000
HD = 128
E = 320000
NW = 32
EW = 10240
EP = EW * NW // 32 * 32
C = 128
NCH = EW // C
NR = 10240
JUNK = N
RPT = NR // 16
EDGE_BLOCK = 2560


def _build_sc_pass(specs, name):
    out_types = []
    scratch_types = []
    in_counts = []
    out_counts = []
    scr_counts = []
    n_row_acc = 0
    n_sca_acc = 0
    for sp in specs:
        k = sp["kind"]
        if k == "gr":
            in_counts.append(2)
            out_types.append(jax.ShapeDtypeStruct((EP, HD), jnp.float32))
            out_counts.append(1)
            scratch_types.extend([pltpu.VMEM((C,), jnp.int32),
                                  pltpu.VMEM((C, HD), jnp.float32)])
            scr_counts.append(2)
        elif k == "ge":
            in_counts.append(2)
            out_types.append(jax.ShapeDtypeStruct((EP,), jnp.float32))
            out_counts.append(1)
            scratch_types.extend([pltpu.VMEM((C,), jnp.int32),
                                  pltpu.VMEM((C,), jnp.float32)])
            scr_counts.append(2)
        elif k == "sr":
            n_row_acc += 1
            in_counts.append(2)
            out_types.append(jax.ShapeDtypeStruct((2 * NR, HD), jnp.float32))
            out_counts.append(1)
            scratch_types.extend([pltpu.VMEM((C,), jnp.int32),
                                  pltpu.VMEM((C, HD), jnp.float32),
                                  pltpu.VMEM_SHARED((NR, HD), jnp.float32)])
            scr_counts.append(3)
        elif k == "se":
            n_sca_acc += 1
            in_counts.append(2)
            out_types.append(jax.ShapeDtypeStruct((2 * NR,), jnp.float32))
            out_counts.append(1)
            scratch_types.extend([pltpu.VMEM((C,), jnp.int32),
                                  pltpu.VMEM((C,), jnp.float32),
                                  pltpu.VMEM_SHARED((NR,), jnp.float32)])
            scr_counts.append(3)
        elif k == "gsr":
            n_row_acc += 1
            in_counts.append(3)
            nout = 1
            out_types.append(jax.ShapeDtypeStruct((2 * NR, HD), jnp.float32))
            if sp.get("lin"):
                out_types.append(jax.ShapeDtypeStruct((EP, HD), jnp.float32))
                nout = 2
            out_counts.append(nout)
            scratch_types.extend([pltpu.VMEM((C,), jnp.int32),
                                  pltpu.VMEM((C,), jnp.int32),
                                  pltpu.VMEM((C, HD), jnp.float32),
                                  pltpu.VMEM_SHARED((NR, HD), jnp.float32)])
            scr_counts.append(4)
        else:
            raise ValueError(k)
    assert n_row_acc <= 1, "only one (NR,HD) Spmem accumulator fits"
    extra = []
    if n_row_acc:
        extra.append(pltpu.VMEM((64, HD), jnp.float32))
    if n_sca_acc:
        extra.append(pltpu.VMEM((RPT,), jnp.float32))
    extra.append(pltpu.SemaphoreType.DMA)
    scratch_types.extend(extra)

    n_in = sum(in_counts)
    n_out = sum(out_counts)

    def body(*refs):
        ins = refs[:n_in]
        outs = refs[n_in:n_in + n_out]
        scrs = refs[n_in + n_out:]
        zrow = scrs[len(scrs) - len(extra)] if n_row_acc else None
        zsca = scrs[len(scrs) - len(extra) + (1 if n_row_acc else 0)] if n_sca_acc else None
        sem = scrs[-1]

        cid = lax.axis_index("c")
        sid = lax.axis_index("s")
        base = (cid * 16 + sid) * EW

        views = []
        io_i = o_i = s_i = 0
        for si, sp in enumerate(specs):
            views.append((ins[io_i:io_i + in_counts[si]],
                          outs[o_i:o_i + out_counts[si]],
                          scrs[s_i:s_i + scr_counts[si]]))
            io_i += in_counts[si]
            o_i += out_counts[si]
            s_i += scr_counts[si]

        if n_row_acc:
            def zr(j, _):
                for kk in range(HD // 16):
                    zrow[j, pl.ds(kk * 16, 16)] = jnp.zeros((16,), jnp.float32)
                return 0
            lax.fori_loop(0, 64, zr, 0)
        if n_sca_acc:
            def zs(j, _):
                zsca[pl.ds(j * 16, 16)] = jnp.zeros((16,), jnp.float32)
                return 0
            lax.fori_loop(0, RPT // 16, zs, 0)
        for si, sp in enumerate(specs):
            _, _, scr = views[si]
            if sp["kind"] in ("sr", "gsr"):
                acc = scr[-1]
                for r in range(RPT // 64):
                    pltpu.sync_copy(zrow, acc.at[pl.ds(sid * RPT + r * 64, 64)])
            elif sp["kind"] == "se":
                acc = scr[-1]
                pltpu.sync_copy(zsca, acc.at[pl.ds(sid * RPT, RPT)])
        if n_row_acc or n_sca_acc:
            plsc.subcore_barrier()

        def chunk(i, _):
            off = base + i * C
            for si, sp in enumerate(specs):
                inr, outr, scr = views[si]
                k = sp["kind"]
                if k == "gr":
                    table, gidx = inr
                    gv, rows = scr
                    pltpu.sync_copy(gidx.at[pl.ds(off, C)], gv)
                    pltpu.async_copy(table.at[gv], rows, sem).wait()
                    pltpu.sync_copy(rows, outr[0].at[pl.ds(off, C)])
                elif k == "ge":
                    table, gidx = inr
                    gv, elems = scr
                    pltpu.sync_copy(gidx.at[pl.ds(off, C)], gv)
                    pltpu.async_copy(table.at[gv], elems, sem).wait()
                    pltpu.sync_copy(elems, outr[0].at[pl.ds(off, C)])
                elif k == "sr":
                    vals, sidx = inr
                    sv, rows, acc = scr
                    pltpu.sync_copy(sidx.at[pl.ds(off, C)], sv)
                    pltpu.sync_copy(vals.at[pl.ds(off, C)], rows)
                    pltpu.sync_copy(rows, acc.at[sv], add=True)
                elif k == "se":
                    vals, sidx = inr
                    sv, elems, acc = scr
                    pltpu.sync_copy(sidx.at[pl.ds(off, C)], sv)
                    pltpu.sync_copy(vals.at[pl.ds(off, C)], elems)
                    pltpu.sync_copy(elems, acc.at[sv], add=True)
                elif k == "gsr":
                    table, gidx, sidx = inr
                    gv, sv, rows, acc = scr
                    pltpu.sync_copy(gidx.at[pl.ds(off, C)], gv)
                    pltpu.async_copy(table.at[gv], rows, sem).wait()
                    pltpu.sync_copy(sidx.at[pl.ds(off, C)], sv)
                    pltpu.sync_copy(rows, acc.at[sv], add=True)
                    if sp.get("lin"):
                        pltpu.sync_copy(rows, outr[1].at[pl.ds(off, C)])
            return 0
        lax.fori_loop(0, NCH, chunk, 0)

        if n_row_acc or n_sca_acc:
            plsc.subcore_barrier()
        for si, sp in enumerate(specs):
            _, outr, scr = views[si]
            if sp["kind"] in ("sr", "gsr"):
                acc = scr[-1]
                pltpu.sync_copy(acc.at[pl.ds(sid * RPT, RPT)],
                                outr[0].at[pl.ds(cid * NR + sid * RPT, RPT)])
            elif sp["kind"] == "se":
                acc = scr[-1]
                pltpu.sync_copy(acc.at[pl.ds(sid * RPT, RPT)],
                                outr[0].at[pl.ds(cid * NR + sid * RPT, RPT)])

    return pl.kernel(
        body,
        out_type=tuple(out_types),
        mesh=plsc.VectorSubcoreMesh(core_axis_name="c", subcore_axis_name="s"),
        scratch_types=tuple(scratch_types),
        name=name,
    )


def _comb_r(p):
    return p[:NR] + p[NR:]


def _comb_s(p):
    return p[:NR] + p[NR:]


def _bdot(a, b):
    return jnp.dot(a.astype(jnp.bfloat16), b.astype(jnp.bfloat16),
                   preferred_element_type=jnp.float32)


def _mlp_u_body(q_ref, go_ref, w0_ref, b0_ref, w1_ref, b1_ref, w2_ref, b2_ref,
                ee_ref, gq_ref):
    q = q_ref[...]
    go = go_ref[...]
    w0 = w0_ref[...]
    w1 = w1_ref[...]
    w2 = w2_ref[...]
    d1 = _bdot(q, w0.T) + b0_ref[0]
    s1 = jnp.tanh(d1)
    d2 = _bdot(s1, w1.T) + b1_ref[0]
    s2 = jnp.maximum(d2, 0.0)
    ee_ref[...] = _bdot(s2, w2.T) + b2_ref[0]
    g_s2 = _bdot(go, w2)
    g_d2 = jnp.where(d2 > 0, g_s2, 0.0)
    g_s1 = _bdot(g_d2, w1)
    g_d1 = g_s1 * (1.0 - s1 * s1)
    gq_ref[...] = _bdot(g_d1, w0)


def _mlp_u_fused(q, g_out, w0, b0, w1, b1, w2, b2):
    grid = (EP // EDGE_BLOCK,)
    row = pl.BlockSpec((EDGE_BLOCK, HD), lambda i: (i, 0))
    wsp = pl.BlockSpec((HD, HD), lambda i: (0, 0))
    bsp = pl.BlockSpec((1, HD), lambda i: (0, 0))
    return pl.pallas_call(
        _mlp_u_body,
        grid=grid,
        in_specs=[row, row, wsp, bsp, wsp, bsp, wsp, bsp],
        out_specs=[row, row],
        out_shape=[jax.ShapeDtypeStruct((EP, HD), jnp.float32),
                   jax.ShapeDtypeStruct((EP, HD), jnp.float32)],
    )(q, g_out, w0, b0.reshape(1, HD), w1, b1.reshape(1, HD), w2,
      b2.reshape(1, HD))


def kernel(x, edge_index, fc_w, attn_w, encK_w, encK_b, encP1_w, encP1_b,
                 encP2_w, encP2_b, K0_w, K0_b, K1_w, K1_b, K2_w, K2_b,
                 U0_w, U0_b, U1_w, U1_b, U2_w, U2_b, D):
    src = edge_index[0].astype(jnp.int32)
    dst = edge_index[1].astype(jnp.int32)
    a1 = attn_w[0, :HD]
    a2 = attn_w[0, HD:]

    npad = EP - E
    pad_g = jnp.zeros((npad,), jnp.int32)
    pad_s = jnp.full((npad,), JUNK, jnp.int32)
    src_g = jnp.concatenate([src, pad_g])
    src_s = jnp.concatenate([src, pad_s])
    dst_g = jnp.concatenate([dst, pad_g])
    dst_s = jnp.concatenate([dst, pad_s])
    ones_e = jnp.ones((EP,), jnp.float32)

    sc1 = _build_sc_pass([
        dict(kind="ge"), dict(kind="ge"),
    ], "sc1_attn_scalars")
    sc2 = _build_sc_pass([dict(kind="gr"), dict(kind="se")], "sc2_zsrc_ssum")
    sc3 = _build_sc_pass([dict(kind="sr"), dict(kind="ge")], "sc3_hemb_sdst")
    sc4a = _build_sc_pass([
        dict(kind="gsr"), dict(kind="gr"),
    ], "sc4a_h1a_p1")
    sc4b = _build_sc_pass([dict(kind="gr")], "sc4b_p2")
    sc5 = _build_sc_pass([dict(kind="gr")], "sc5_enode_src")
    sc6 = _build_sc_pass([dict(kind="sr")], "sc6_eedge_src")
    sc7 = _build_sc_pass([dict(kind="sr")], "sc7_gq_src")
    sc8 = _build_sc_pass([dict(kind="sr")], "sc8_gq_dst")
    sc9 = _build_sc_pass([dict(kind="gsr")], "sc9_gh1a")
    sc10 = _build_sc_pass([dict(kind="gr")], "sc10_ghemb_dst")
    sc11 = _build_sc_pass([dict(kind="sr"), dict(kind="se")], "sc11_gz_c")
    sc12 = _build_sc_pass([dict(kind="ge")], "sc12_c_dst")
    sc13 = _build_sc_pass([dict(kind="se"), dict(kind="se")], "sc13_su_du")

    z = x @ fc_w.T
    za1 = z @ a1
    za2 = z @ a2

    ea, eb = sc1(za1, src_g, za2, dst_g)
    e = jax.nn.sigmoid(ea + eb)
    ex = jnp.exp(e)

    zs, sp_ = sc2(z, src_g, ex, dst_s)
    s_full = _comb_s(sp_)
    zs_scaled = ex[:, None] * zs
    hembp, sdst = sc3(zs_scaled, dst_s, s_full, dst_g)
    s_n = s_full[:N]
    s_safe = jnp.where(s_n > 0, s_n, 1.0)
    h_emb = _comb_r(hembp)[:N] / s_safe[:, None]

    h1 = h_emb @ encK_w.T + encK_b
    p1 = h_emb @ encP1_w.T + encP1_b
    p2 = h_emb @ encP2_w.T + encP2_b
    h1ap, p1s = sc4a(h1, src_g, dst_s, p1, src_g)
    (p2d,) = sc4b(p2, dst_g)
    h1a = _comb_r(h1ap)[:N]
    c1 = h1a @ K0_w.T + K0_b
    t1 = jnp.tanh(c1)
    c2 = t1 @ K1_w.T + K1_b
    t2 = jax.nn.relu(c2)
    E_node = t2 @ K2_w.T + K2_b

    (gEedge,) = sc5(E_node, src_g)

    q = p1s + p2d
    E_edge, g_q = _mlp_u_fused(q, gEedge, U0_w, U0_b, U1_w, U1_b, U2_w, U2_b)

    (eesump,) = sc6(E_edge, src_s)
    (gqsp,) = sc7(g_q, src_s)
    (gqdp,) = sc8(g_q, dst_s)
    gEnode = _comb_r(eesump)[:N]
    g_p1 = _comb_r(gqsp)[:N]
    g_p2 = _comb_r(gqdp)[:N]

    g_t2 = gEnode @ K2_w
    g_c2 = g_t2 * (c2 > 0)
    g_t1 = g_c2 @ K1_w
    g_c1 = g_t1 * (1.0 - t1 * t1)
    g_h1a = g_c1 @ K0_w

    (gh1p,) = sc9(g_h1a, dst_g, src_s)
    Gh1 = _comb_r(gh1p)[:N]
    g_hemb = Gh1 @ encK_w + g_p1 @ encP1_w + g_p2 @ encP2_w

    (gd,) = sc10(g_hemb, dst_g)
    sdst_safe = jnp.where(sdst > 0, sdst, 1.0)
    alpha = ex / sdst_safe
    g_alpha = jnp.sum(gd * zs, axis=1)
    scaled = alpha[:, None] * gd
    cvals = alpha * g_alpha
    gzp, cp = sc11(scaled, src_s, cvals, dst_s)
    GZ = _comb_r(gzp)[:N]
    c_full = _comb_s(cp)
    (cd,) = sc12(c_full, dst_g)
    g_u = alpha * (g_alpha - cd) * e * (1.0 - e)
    sup, dup = sc13(g_u, src_s, g_u, dst_s)
    su = _comb_s(sup)[:N]
    du = _comb_s(dup)[:N]

    g_z = GZ + su[:, None] * a1[None, :] + du[:, None] * a2[None, :]
    g_x = g_z @ fc_w
    d = x.shape[1]
    M = jnp.eye(d, dtype=x.dtype)
    J = jnp.concatenate([M[d // 2:], -M[:d // 2]], axis=0)
    return g_x @ J.T - g_x @ D.T


# speedup vs baseline: 1.5395x; 1.5395x over previous
import jax
import jax.numpy as jnp
from jax import lax
from jax.experimental import pallas as pl
from jax.experimental.pallas import tpu as pltpu
from jax.experimental.pallas import tpu_sc as plsc

N = 10000
HD = 128
E = 320000
NW = 32
EW = 10240
EP = EW * NW // 32 * 32
C = 128
NCH = EW // C
NR = 10240
JUNK = N
RPT = NR // 16
EDGE_BLOCK = 2560


def _build_sc_pass(specs, name):
    out_types = []
    scratch_types = []
    in_counts = []
    out_counts = []
    scr_counts = []
    has_row_acc = any(sp["kind"] in ("sr", "gsr") for sp in specs)
    DEPTH = 2 if has_row_acc else 3
    GB = 16 if has_row_acc else 80
    NGRP = NCH // GB
    n_row_acc = 0
    n_sca_acc = 0
    for sp in specs:
        k = sp["kind"]
        sems = [pltpu.SemaphoreType.DMA] * (2 * DEPTH)
        if k == "gr":
            in_counts.append(2)
            out_types.append(jax.ShapeDtypeStruct((EP, HD), jnp.float32))
            out_counts.append(1)
            scratch_types.extend(
                [pltpu.VMEM((GB, C), jnp.int32)]
                + [pltpu.VMEM((C, HD), jnp.float32)] * DEPTH + sems)
            scr_counts.append(1 + 3 * DEPTH)
        elif k == "ge":
            in_counts.append(2)
            out_types.append(jax.ShapeDtypeStruct((EP,), jnp.float32))
            out_counts.append(1)
            scratch_types.extend(
                [pltpu.VMEM((GB, C), jnp.int32)]
                + [pltpu.VMEM((C,), jnp.float32)] * DEPTH + sems)
            scr_counts.append(1 + 3 * DEPTH)
        elif k in ("sr", "se"):
            shape = (C, HD) if k == "sr" else (C,)
            acc_shape = (NR, HD) if k == "sr" else (NR,)
            if k == "sr":
                n_row_acc += 1
            else:
                n_sca_acc += 1
            in_counts.append(2)
            out_types.append(jax.ShapeDtypeStruct(
                (2 * NR, HD) if k == "sr" else (2 * NR,), jnp.float32))
            out_counts.append(1)
            scratch_types.extend(
                [pltpu.VMEM((GB, C), jnp.int32)]
                + [pltpu.VMEM(shape, jnp.float32)] * DEPTH
                + [pltpu.VMEM_SHARED(acc_shape, jnp.float32)] + sems)
            scr_counts.append(2 + 3 * DEPTH)
        elif k == "gsr":
            n_row_acc += 1
            in_counts.append(3)
            out_types.append(jax.ShapeDtypeStruct((2 * NR, HD), jnp.float32))
            out_counts.append(1)
            scratch_types.extend(
                [pltpu.VMEM((GB, C), jnp.int32), pltpu.VMEM((GB, C), jnp.int32)]
                + [pltpu.VMEM((C, HD), jnp.float32)] * DEPTH
                + [pltpu.VMEM_SHARED((NR, HD), jnp.float32)] + sems)
            scr_counts.append(3 + 3 * DEPTH)
        else:
            raise ValueError(k)
    assert n_row_acc <= 1, "only one (NR,HD) Spmem accumulator fits"
    extra = []
    if n_row_acc:
        extra.append(pltpu.VMEM((64, HD), jnp.float32))
    if n_sca_acc:
        extra.append(pltpu.VMEM((RPT,), jnp.float32))
    scratch_types.extend(extra)

    n_in = sum(in_counts)
    n_out = sum(out_counts)
    n_extra = len(extra)

    def body(*refs):
        ins = refs[:n_in]
        outs = refs[n_in:n_in + n_out]
        scrs = refs[n_in + n_out:]
        zrow = scrs[len(scrs) - n_extra] if n_row_acc else None
        zsca = scrs[len(scrs) - n_extra + (1 if n_row_acc else 0)] if n_sca_acc else None

        cid = lax.axis_index("c")
        sid = lax.axis_index("s")
        wid = cid * 16 + sid

        views = []
        io_i = o_i = s_i = 0
        for si, sp in enumerate(specs):
            views.append((ins[io_i:io_i + in_counts[si]],
                          outs[o_i:o_i + out_counts[si]],
                          scrs[s_i:s_i + scr_counts[si]]))
            io_i += in_counts[si]
            o_i += out_counts[si]
            s_i += scr_counts[si]

        if n_row_acc:
            def zr(j, _):
                for kk in range(HD // 16):
                    zrow[j, pl.ds(kk * 16, 16)] = jnp.zeros((16,), jnp.float32)
                return 0
            lax.fori_loop(0, 64, zr, 0)
        if n_sca_acc:
            def zsf(j, _):
                zsca[pl.ds(j * 16, 16)] = jnp.zeros((16,), jnp.float32)
                return 0
            lax.fori_loop(0, RPT // 16, zsf, 0)
        for si, sp in enumerate(specs):
            _, _, scr = views[si]
            if sp["kind"] in ("sr", "gsr"):
                acc = scr[-(2 * DEPTH + 1)]
                for r in range(RPT // 64):
                    pltpu.sync_copy(zrow, acc.at[pl.ds(sid * RPT + r * 64, 64)])
            elif sp["kind"] == "se":
                acc = scr[-(2 * DEPTH + 1)]
                pltpu.sync_copy(zsca, acc.at[pl.ds(sid * RPT, RPT)])
        if n_row_acc or n_sca_acc:
            plsc.subcore_barrier()

        def group(g, _):
            row0 = wid * NCH + g * GB
            for si, sp in enumerate(specs):
                inr, _, scr = views[si]
                pltpu.sync_copy(inr[1].at[pl.ds(row0, GB)], scr[0])
                if sp["kind"] == "gsr":
                    pltpu.sync_copy(inr[2].at[pl.ds(row0, GB)], scr[1])

            st = [{"a": [None] * DEPTH, "b": [None] * DEPTH} for _ in specs]

            def start_load(si, j):
                inr, outr, scr = views[si]
                k = specs[si]["kind"]
                b = j % DEPTH
                if k in ("gr", "ge"):
                    st[si]["a"][b] = pltpu.async_copy(
                        inr[0].at[scr[0].at[j]], scr[1 + b], scr[1 + DEPTH + b])
                elif k in ("sr", "se"):
                    off = (row0 + j) * C
                    st[si]["a"][b] = pltpu.async_copy(
                        inr[0].at[pl.ds(off, C)], scr[1 + b],
                        scr[2 + DEPTH + b])
                elif k == "gsr":
                    st[si]["a"][b] = pltpu.async_copy(
                        inr[0].at[scr[0].at[j]], scr[2 + b],
                        scr[3 + DEPTH + b])

            def start_store(si, j):
                inr, outr, scr = views[si]
                k = specs[si]["kind"]
                b = j % DEPTH
                if k in ("gr", "ge"):
                    off = (row0 + j) * C
                    st[si]["b"][b] = pltpu.async_copy(
                        scr[1 + b], outr[0].at[pl.ds(off, C)],
                        scr[1 + 2 * DEPTH + b])
                elif k in ("sr", "se"):
                    st[si]["b"][b] = pltpu.async_copy(
                        scr[1 + b], scr[1 + DEPTH].at[scr[0].at[j]],
                        scr[2 + 2 * DEPTH + b], add=True)
                elif k == "gsr":
                    st[si]["b"][b] = pltpu.async_copy(
                        scr[2 + b], scr[2 + DEPTH].at[scr[1].at[j]],
                        scr[3 + 2 * DEPTH + b], add=True)

            for si in range(len(specs)):
                for k0 in range(DEPTH - 1):
                    start_load(si, k0)
            for j in range(GB):
                b = j % DEPTH
                for si in range(len(specs)):
                    st[si]["a"][b].wait()
                    nxt = j + DEPTH - 1
                    if nxt < GB:
                        if nxt - DEPTH >= 0:
                            st[si]["b"][nxt % DEPTH].wait()
                        start_load(si, nxt)
                    start_store(si, j)
            for si in range(len(specs)):
                for j in range(max(0, GB - DEPTH), GB):
                    st[si]["b"][j % DEPTH].wait()
            return 0
        lax.fori_loop(0, NGRP, group, 0)

        if n_row_acc or n_sca_acc:
            plsc.subcore_barrier()
        for si, sp in enumerate(specs):
            _, outr, scr = views[si]
            if sp["kind"] in ("sr", "se", "gsr"):
                acc = scr[-(2 * DEPTH + 1)]
                pltpu.sync_copy(acc.at[pl.ds(sid * RPT, RPT)],
                                outr[0].at[pl.ds(cid * NR + sid * RPT, RPT)])

    return pl.kernel(
        body,
        out_type=tuple(out_types),
        mesh=plsc.VectorSubcoreMesh(core_axis_name="c", subcore_axis_name="s"),
        scratch_types=tuple(scratch_types),
        name=name,
    )


def _comb_r(p):
    return p[:NR] + p[NR:]


def _comb_s(p):
    return p[:NR] + p[NR:]


def _bdot(a, b):
    return jnp.dot(a.astype(jnp.bfloat16), b.astype(jnp.bfloat16),
                   preferred_element_type=jnp.float32)


def _mlp_u_body(p1_ref, p2_ref, go_ref, w0_ref, b0_ref, w1_ref, b1_ref,
                w2_ref, b2_ref, ee_ref, gq_ref):
    q = p1_ref[...] + p2_ref[...]
    go = go_ref[...]
    w0 = w0_ref[...]
    w1 = w1_ref[...]
    w2 = w2_ref[...]
    d1 = _bdot(q, w0.T) + b0_ref[0]
    s1 = jnp.tanh(d1)
    d2 = _bdot(s1, w1.T) + b1_ref[0]
    s2 = jnp.maximum(d2, 0.0)
    ee_ref[...] = _bdot(s2, w2.T) + b2_ref[0]
    g_s2 = _bdot(go, w2)
    g_d2 = jnp.where(d2 > 0, g_s2, 0.0)
    g_s1 = _bdot(g_d2, w1)
    g_d1 = g_s1 * (1.0 - s1 * s1)
    gq_ref[...] = _bdot(g_d1, w0)


def _mlp_u_fused(p1s, p2d, g_out, w0, b0, w1, b1, w2, b2):
    grid = (EP // EDGE_BLOCK,)
    row = pl.BlockSpec((EDGE_BLOCK, HD), lambda i: (i, 0))
    wsp = pl.BlockSpec((HD, HD), lambda i: (0, 0))
    bsp = pl.BlockSpec((1, HD), lambda i: (0, 0))
    return pl.pallas_call(
        _mlp_u_body,
        grid=grid,
        in_specs=[row, row, row, wsp, bsp, wsp, bsp, wsp, bsp],
        out_specs=[row, row],
        out_shape=[jax.ShapeDtypeStruct((EP, HD), jnp.float32),
                   jax.ShapeDtypeStruct((EP, HD), jnp.float32)],
    )(p1s, p2d, g_out, w0, b0.reshape(1, HD), w1, b1.reshape(1, HD), w2,
      b2.reshape(1, HD))


def kernel(x, edge_index, fc_w, attn_w, encK_w, encK_b, encP1_w, encP1_b,
                 encP2_w, encP2_b, K0_w, K0_b, K1_w, K1_b, K2_w, K2_b,
                 U0_w, U0_b, U1_w, U1_b, U2_w, U2_b, D):
    src = edge_index[0].astype(jnp.int32)
    dst = edge_index[1].astype(jnp.int32)
    a1 = attn_w[0, :HD]
    a2 = attn_w[0, HD:]

    npad = EP - E
    pad_g = jnp.zeros((npad,), jnp.int32)
    pad_s = jnp.full((npad,), JUNK, jnp.int32)
    src_g = jnp.concatenate([src, pad_g]).reshape(EP // C, C)
    src_s = jnp.concatenate([src, pad_s]).reshape(EP // C, C)
    dst_g = jnp.concatenate([dst, pad_g]).reshape(EP // C, C)
    dst_s = jnp.concatenate([dst, pad_s]).reshape(EP // C, C)

    sc1 = _build_sc_pass([
        dict(kind="ge"), dict(kind="ge"),
    ], "sc1_attn_scalars")
    sc2 = _build_sc_pass([dict(kind="gr"), dict(kind="se")], "sc2_zsrc_ssum")
    sc3 = _build_sc_pass([dict(kind="sr"), dict(kind="ge")], "sc3_hemb_sdst")
    sc4a = _build_sc_pass([dict(kind="gsr")], "sc4a_h1a")
    sc4b = _build_sc_pass([dict(kind="gr"), dict(kind="gr")], "sc4b_p1p2")
    sc5 = _build_sc_pass([dict(kind="gr")], "sc5_enode_src")
    sc6 = _build_sc_pass([dict(kind="sr")], "sc6_eedge_src")
    sc7 = _build_sc_pass([dict(kind="sr")], "sc7_gq_src")
    sc8 = _build_sc_pass([dict(kind="sr")], "sc8_gq_dst")
    sc9 = _build_sc_pass([dict(kind="gsr")], "sc9_gh1a")
    sc10 = _build_sc_pass([dict(kind="gr")], "sc10_ghemb_dst")
    sc11 = _build_sc_pass([dict(kind="sr"), dict(kind="se")], "sc11_gz_c")
    sc12 = _build_sc_pass([dict(kind="ge")], "sc12_c_dst")
    sc13 = _build_sc_pass([dict(kind="se"), dict(kind="se")], "sc13_su_du")

    z = x @ fc_w.T
    za1 = z @ a1
    za2 = z @ a2

    ea, eb = sc1(za1, src_g, za2, dst_g)
    e = jax.nn.sigmoid(ea + eb)
    ex = jnp.exp(e)

    zs, sp_ = sc2(z, src_g, ex, dst_s)
    s_full = _comb_s(sp_)
    zs_scaled = ex[:, None] * zs
    hembp, sdst = sc3(zs_scaled, dst_s, s_full, dst_g)
    s_n = s_full[:N]
    s_safe = jnp.where(s_n > 0, s_n, 1.0)
    h_emb = _comb_r(hembp)[:N] / s_safe[:, None]

    h1 = h_emb @ encK_w.T + encK_b
    p1 = h_emb @ encP1_w.T + encP1_b
    p2 = h_emb @ encP2_w.T + encP2_b
    (h1ap,) = sc4a(h1, src_g, dst_s)
    p1s, p2d = sc4b(p1, src_g, p2, dst_g)
    h1a = _comb_r(h1ap)[:N]
    c1 = h1a @ K0_w.T + K0_b
    t1 = jnp.tanh(c1)
    c2 = t1 @ K1_w.T + K1_b
    t2 = jax.nn.relu(c2)
    E_node = t2 @ K2_w.T + K2_b

    (gEedge,) = sc5(E_node, src_g)

    E_edge, g_q = _mlp_u_fused(p1s, p2d, gEedge, U0_w, U0_b, U1_w, U1_b,
                               U2_w, U2_b)

    (eesump,) = sc6(E_edge, src_s)
    (gqsp,) = sc7(g_q, src_s)
    (gqdp,) = sc8(g_q, dst_s)
    gEnode = _comb_r(eesump)[:N]
    g_p1 = _comb_r(gqsp)[:N]
    g_p2 = _comb_r(gqdp)[:N]

    g_t2 = gEnode @ K2_w
    g_c2 = g_t2 * (c2 > 0)
    g_t1 = g_c2 @ K1_w
    g_c1 = g_t1 * (1.0 - t1 * t1)
    g_h1a = g_c1 @ K0_w

    (gh1p,) = sc9(g_h1a, dst_g, src_s)
    Gh1 = _comb_r(gh1p)[:N]
    g_hemb = Gh1 @ encK_w + g_p1 @ encP1_w + g_p2 @ encP2_w

    (gd,) = sc10(g_hemb, dst_g)
    sdst_safe = jnp.where(sdst > 0, sdst, 1.0)
    alpha = ex / sdst_safe
    g_alpha = jnp.sum(gd * zs, axis=1)
    scaled = alpha[:, None] * gd
    cvals = alpha * g_alpha
    gzp, cp = sc11(scaled, src_s, cvals, dst_s)
    GZ = _comb_r(gzp)[:N]
    c_full = _comb_s(cp)
    (cd,) = sc12(c_full, dst_g)
    g_u = alpha * (g_alpha - cd) * e * (1.0 - e)
    sup, dup = sc13(g_u, src_s, g_u, dst_s)
    su = _comb_s(sup)[:N]
    du = _comb_s(dup)[:N]

    g_z = GZ + su[:, None] * a1[None, :] + du[:, None] * a2[None, :]
    g_x = g_z @ fc_w
    d = x.shape[1]
    M = jnp.eye(d, dtype=x.dtype)
    J = jnp.concatenate([M[d // 2:], -M[:d // 2]], axis=0)
    return g_x @ J.T - g_x @ D.T
